# baseline (speedup 1.0000x reference)
import functools

import jax
import jax.numpy as jnp
from jax import lax
from jax.experimental import pallas as pl
from jax.experimental.pallas import tpu as pltpu
from jax.experimental.pallas import tpu_sc as plsc

_N = 4096
_D = 128
_C = 64
_L = 16
_NC = 2
_W = 16
_B = 128
_NCH = _D // _L
_TR = _C + _C + _C + _W


def _rsqrt16(x):
    x = jnp.maximum(x, jnp.full((_L,), 1e-35, jnp.float32))
    i = jnp.full((_L,), 0x5F3759DF, jnp.int32) - lax.shift_right_logical(
        plsc.bitcast(x, jnp.int32), jnp.full((_L,), 1, jnp.int32)
    )
    y = plsc.bitcast(i, jnp.float32)
    xh = x * jnp.float32(0.5)
    for _ in range(2):
        y = y * (jnp.float32(1.5) - xh * y * y)
    return y


def _lanesum(v):
    for s in (1, 2, 4, 8):
        idx = lax.iota(jnp.int32, _L) ^ s
        v = v + jnp.take_along_axis(v, idx, axis=0)
    return v


def _sc_body(ps_hbm, zs_hbm, tg_hbm, out_hbm, pv, zv, tv, ones_v, zb, zcb,
             dbuf, shp, shz, shm, dma_sem):
    ci = lax.axis_index("c")
    w = lax.axis_index("s")
    g = ci * _W + w
    zero16 = jnp.zeros((_L,), jnp.float32)
    one16 = jnp.ones((_L,), jnp.float32)

    cp_p = pltpu.async_copy(ps_hbm.at[g], pv, dma_sem)
    cp_z = pltpu.async_copy(zs_hbm.at[g], zv, dma_sem)
    cp_t = pltpu.async_copy(tg_hbm.at[g], tv, dma_sem)

    for r in range(4):
        for c in range(_NCH):
            zb[r, pl.ds(c * _L, _L)] = zero16
    for r in range(5):
        for c in range(_NCH):
            zcb[r, pl.ds(c * _L, _L)] = zero16
    for c in range(_NCH):
        dbuf[pl.ds(c * _L, _L)] = zero16

    @plsc.parallel_loop(0, _B)
    def _ones_row(r):
        for c in range(_NCH):
            ones_v[r, pl.ds(c * _L, _L)] = one16

    pltpu.sync_copy(zb, shp.at[pl.ds(w * 4, 4)])
    pltpu.sync_copy(zb, shz.at[pl.ds(w * 4, 4)])
    pltpu.sync_copy(zcb, shm.at[pl.ds(w * 5, 5)])
    plsc.subcore_barrier()

    with jax.named_scope("stage_wait"):
        cp_p.wait()
        cp_z.wait()
        cp_t.wait()

    _ns_norm = jax.named_scope("normalize"); _ns_norm.__enter__()

    @plsc.parallel_loop(0, _B, unroll=2, carry=(zero16,) * _NCH)
    def daccs(r, ds_c):
        pc = [pv[r, pl.ds(c * _L, _L)] for c in range(_NCH)]
        zc = [zv[r, pl.ds(c * _L, _L)] for c in range(_NCH)]
        ssp = pc[0] * pc[0]
        ssz = zc[0] * zc[0]
        for c in range(1, _NCH):
            ssp += pc[c] * pc[c]
            ssz += zc[c] * zc[c]
        sp = _lanesum(ssp)
        sz = _lanesum(ssz)
        ip = jnp.minimum(_rsqrt16(sp), jnp.full((_L,), 1e8, jnp.float32))
        iz = jnp.minimum(_rsqrt16(sz), jnp.full((_L,), 1e8, jnp.float32))
        out = []
        for c in range(_NCH):
            pn = pc[c] * ip
            zn = zc[c] * iz
            pv[r, pl.ds(c * _L, _L)] = pn
            zv[r, pl.ds(c * _L, _L)] = zn
            out.append(ds_c[c] + pn * zn)
        return tuple(out)

    dacc = daccs[0]
    for c in range(1, _NCH):
        dacc = dacc + daccs[c]
    _ns_norm.__exit__(None, None, None)

    with jax.named_scope("scatter"):
        sc_p = pltpu.async_copy(pv, shp.at[tv], dma_sem, add=True)
        sc_z = pltpu.async_copy(zv, shz.at[tv], dma_sem, add=True)
        sc_o = pltpu.async_copy(ones_v, shm.at[tv], dma_sem, add=True)
        dbuf[pl.ds(0, _L)] = dacc
        sc_p.wait()
        sc_z.wait()
        sc_o.wait()
        pltpu.sync_copy(dbuf, shm.at[_C + w])
        plsc.subcore_barrier()

    with jax.named_scope("dump"):
        q = _C // 4
        @pl.when(w < 4)
        def _dump_p():
            pltpu.sync_copy(shp.at[pl.ds(w * q, q)],
                            out_hbm.at[ci, pl.ds(w * q, q)])
        @pl.when((w >= 4) & (w < 8))
        def _dump_z():
            pltpu.sync_copy(shz.at[pl.ds((w - 4) * q, q)],
                            out_hbm.at[ci, pl.ds(_C + (w - 4) * q, q)])
        @pl.when((w >= 8) & (w < 13))
        def _dump_m():
            m = 16
            pltpu.sync_copy(shm.at[pl.ds((w - 8) * m, m)],
                            out_hbm.at[ci, pl.ds(2 * _C + (w - 8) * m, m)])


_mesh = plsc.VectorSubcoreMesh(
    core_axis_name="c", subcore_axis_name="s", num_cores=_NC, num_subcores=_W
)

_simsiam_sc = functools.partial(
    pl.kernel,
    out_type=jax.ShapeDtypeStruct((_NC, _TR, _D), jnp.float32),
    mesh=_mesh,
    compiler_params=pltpu.CompilerParams(needs_layout_passes=False),
    scratch_types=[
        pltpu.VMEM((_B, _D), jnp.float32),
        pltpu.VMEM((_B, _D), jnp.float32),
        pltpu.VMEM((_B,), jnp.int32),
        pltpu.VMEM((_B, _D), jnp.float32),
        pltpu.VMEM((4, _D), jnp.float32),
        pltpu.VMEM((5, _D), jnp.float32),
        pltpu.VMEM((_D,), jnp.float32),
        pltpu.VMEM_SHARED((_C, _D), jnp.float32),
        pltpu.VMEM_SHARED((_C, _D), jnp.float32),
        pltpu.VMEM_SHARED((_C + _W, _D), jnp.float32),
        pltpu.SemaphoreType.DMA,
    ],
)(_sc_body)


def _tc_combine_body(t_ref, out_ref):
    sp = t_ref[0, 0:_C, :] + t_ref[1, 0:_C, :]
    sz = t_ref[0, _C:2 * _C, :] + t_ref[1, _C:2 * _C, :]
    cnt = t_ref[0, 2 * _C:3 * _C, :] + t_ref[1, 2 * _C:3 * _C, :]
    dg = t_ref[0, 3 * _C:, :] + t_ref[1, 3 * _C:, :]
    s_all = jnp.sum(sp * sz)
    d_all = jnp.sum(dg)
    nc = cnt[:, 0:1]
    pairs = jnp.sum(nc * nc) - jnp.float32(_N)
    out_ref[0, 0] = -(s_all - d_all) / pairs


_tc_combine = pl.pallas_call(
    _tc_combine_body,
    out_shape=jax.ShapeDtypeStruct((1, 1), jnp.float32),
    out_specs=pl.BlockSpec(memory_space=pltpu.MemorySpace.SMEM),
)


def kernel(ps, zs, targets):
    ps_r = ps.reshape(_NC * _W, _B, _D)
    zs_r = zs.reshape(_NC * _W, _B, _D)
    tg_r = targets.reshape(_NC * _W, _B).astype(jnp.int32)
    tabs = _simsiam_sc(ps_r, zs_r, tg_r)
    return _tc_combine(tabs).reshape(())


# speedup vs baseline: 1.0980x; 1.0980x over previous
import functools

import jax
import jax.numpy as jnp
from jax import lax
from jax.experimental import pallas as pl
from jax.experimental.pallas import tpu as pltpu
from jax.experimental.pallas import tpu_sc as plsc

_N = 4096
_D = 128
_C = 64
_L = 16
_NC = 2
_W = 16
_B = 128
_NCH = _D // _L
_TR = _C + _C + _C + _W


def _rsqrt16(x):
    x = jnp.maximum(x, jnp.full((_L,), 1e-35, jnp.float32))
    i = jnp.full((_L,), 0x5F3759DF, jnp.int32) - lax.shift_right_logical(
        plsc.bitcast(x, jnp.int32), jnp.full((_L,), 1, jnp.int32)
    )
    y = plsc.bitcast(i, jnp.float32)
    xh = x * jnp.float32(0.5)
    for _ in range(2):
        y = y * (jnp.float32(1.5) - xh * y * y)
    return y


def _lanesum(v):
    for s in (1, 2, 4, 8):
        idx = lax.iota(jnp.int32, _L) ^ s
        v = v + jnp.take_along_axis(v, idx, axis=0)
    return v


def _sc_body(ps_hbm, zs_hbm, tg_hbm, out_hbm, pv, zv, tv, ones_v, zb, zcb,
             dbuf, shp, shz, shm, dma_sem):
    ci = lax.axis_index("c")
    w = lax.axis_index("s")
    g = ci * _W + w
    zero16 = jnp.zeros((_L,), jnp.float32)
    one16 = jnp.ones((_L,), jnp.float32)

    cp_p = pltpu.async_copy(ps_hbm.at[g], pv, dma_sem)
    cp_z = pltpu.async_copy(zs_hbm.at[g], zv, dma_sem)
    cp_t = pltpu.async_copy(tg_hbm.at[g], tv, dma_sem)

    for r in range(4):
        for c in range(_NCH):
            zb[r, pl.ds(c * _L, _L)] = zero16
    for r in range(5):
        for c in range(_NCH):
            zcb[r, pl.ds(c * _L, _L)] = zero16
    for c in range(_NCH):
        dbuf[pl.ds(c * _L, _L)] = zero16

    @plsc.parallel_loop(0, _B)
    def _ones_row(r):
        for c in range(_NCH):
            ones_v[r, pl.ds(c * _L, _L)] = one16

    pltpu.sync_copy(zb, shp.at[pl.ds(w * 4, 4)])
    pltpu.sync_copy(zb, shz.at[pl.ds(w * 4, 4)])
    pltpu.sync_copy(zcb, shm.at[pl.ds(w * 5, 5)])
    plsc.subcore_barrier()

    with jax.named_scope("stage_wait"):
        cp_p.wait()
        cp_z.wait()
        cp_t.wait()

    _ns_norm = jax.named_scope("normalize"); _ns_norm.__enter__()

    @plsc.parallel_loop(0, _B, carry=(zero16,) * _NCH)
    def daccs(r, ds_c):
        pc = [pv[r, pl.ds(c * _L, _L)] for c in range(_NCH)]
        zc = [zv[r, pl.ds(c * _L, _L)] for c in range(_NCH)]
        ssp = pc[0] * pc[0]
        ssz = zc[0] * zc[0]
        for c in range(1, _NCH):
            ssp += pc[c] * pc[c]
            ssz += zc[c] * zc[c]
        sp = _lanesum(ssp)
        sz = _lanesum(ssz)
        ip = jnp.minimum(_rsqrt16(sp), jnp.full((_L,), 1e8, jnp.float32))
        iz = jnp.minimum(_rsqrt16(sz), jnp.full((_L,), 1e8, jnp.float32))
        out = []
        for c in range(_NCH):
            pn = pc[c] * ip
            zn = zc[c] * iz
            pv[r, pl.ds(c * _L, _L)] = pn
            zv[r, pl.ds(c * _L, _L)] = zn
            out.append(ds_c[c] + pn * zn)
        return tuple(out)

    dacc = daccs[0]
    for c in range(1, _NCH):
        dacc = dacc + daccs[c]
    _ns_norm.__exit__(None, None, None)

    with jax.named_scope("scatter"):
        sc_p = pltpu.async_copy(pv, shp.at[tv], dma_sem, add=True)
        sc_z = pltpu.async_copy(zv, shz.at[tv], dma_sem, add=True)
        sc_o = pltpu.async_copy(ones_v, shm.at[tv], dma_sem, add=True)
        dbuf[pl.ds(0, _L)] = dacc
        sc_p.wait()
        sc_z.wait()
        sc_o.wait()
        pltpu.sync_copy(dbuf, shm.at[_C + w])
        plsc.subcore_barrier()

    with jax.named_scope("dump"):
        q = _C // 4
        @pl.when(w < 4)
        def _dump_p():
            pltpu.sync_copy(shp.at[pl.ds(w * q, q)],
                            out_hbm.at[ci, pl.ds(w * q, q)])
        @pl.when((w >= 4) & (w < 8))
        def _dump_z():
            pltpu.sync_copy(shz.at[pl.ds((w - 4) * q, q)],
                            out_hbm.at[ci, pl.ds(_C + (w - 4) * q, q)])
        @pl.when((w >= 8) & (w < 13))
        def _dump_m():
            m = 16
            pltpu.sync_copy(shm.at[pl.ds((w - 8) * m, m)],
                            out_hbm.at[ci, pl.ds(2 * _C + (w - 8) * m, m)])


_mesh = plsc.VectorSubcoreMesh(
    core_axis_name="c", subcore_axis_name="s", num_cores=_NC, num_subcores=_W
)

_simsiam_sc = functools.partial(
    pl.kernel,
    out_type=jax.ShapeDtypeStruct((_NC, _TR, _D), jnp.float32),
    mesh=_mesh,
    compiler_params=pltpu.CompilerParams(needs_layout_passes=False),
    scratch_types=[
        pltpu.VMEM((_B, _D), jnp.float32),
        pltpu.VMEM((_B, _D), jnp.float32),
        pltpu.VMEM((_B,), jnp.int32),
        pltpu.VMEM((_B, _D), jnp.float32),
        pltpu.VMEM((4, _D), jnp.float32),
        pltpu.VMEM((5, _D), jnp.float32),
        pltpu.VMEM((_D,), jnp.float32),
        pltpu.VMEM_SHARED((_C, _D), jnp.float32),
        pltpu.VMEM_SHARED((_C, _D), jnp.float32),
        pltpu.VMEM_SHARED((_C + _W, _D), jnp.float32),
        pltpu.SemaphoreType.DMA,
    ],
)(_sc_body)


def _tc_combine_body(t_ref, out_ref):
    sp = t_ref[0, 0:_C, :] + t_ref[1, 0:_C, :]
    sz = t_ref[0, _C:2 * _C, :] + t_ref[1, _C:2 * _C, :]
    cnt = t_ref[0, 2 * _C:3 * _C, :] + t_ref[1, 2 * _C:3 * _C, :]
    dg = t_ref[0, 3 * _C:, :] + t_ref[1, 3 * _C:, :]
    s_all = jnp.sum(sp * sz)
    d_all = jnp.sum(dg)
    nc = cnt[:, 0:1]
    pairs = jnp.sum(nc * nc) - jnp.float32(_N)
    out_ref[0, 0] = -(s_all - d_all) / pairs


_tc_combine = pl.pallas_call(
    _tc_combine_body,
    out_shape=jax.ShapeDtypeStruct((1, 1), jnp.float32),
    out_specs=pl.BlockSpec(memory_space=pltpu.MemorySpace.SMEM),
)


def kernel(ps, zs, targets):
    ps_r = ps.reshape(_NC * _W, _B, _D)
    zs_r = zs.reshape(_NC * _W, _B, _D)
    tg_r = targets.reshape(_NC * _W, _B).astype(jnp.int32)
    tabs = _simsiam_sc(ps_r, zs_r, tg_r)
    return _tc_combine(tabs).reshape(())


# speedup vs baseline: 1.1223x; 1.0221x over previous
import functools

import jax
import jax.numpy as jnp
from jax import lax
from jax.experimental import pallas as pl
from jax.experimental.pallas import tpu as pltpu
from jax.experimental.pallas import tpu_sc as plsc

_N = 4096
_D = 128
_C = 64
_L = 16
_NC = 2
_W = 16
_B = 128
_H = 64
_NCH = _D // _L
_TR = _C + _C + _C + _W


def _rsqrt16(x):
    x = jnp.maximum(x, jnp.full((_L,), 1e-35, jnp.float32))
    i = jnp.full((_L,), 0x5F3759DF, jnp.int32) - lax.shift_right_logical(
        plsc.bitcast(x, jnp.int32), jnp.full((_L,), 1, jnp.int32)
    )
    y = plsc.bitcast(i, jnp.float32)
    xh = x * jnp.float32(0.5)
    for _ in range(2):
        y = y * (jnp.float32(1.5) - xh * y * y)
    return y


def _lanesum(v):
    for s in (1, 2, 4, 8):
        idx = lax.iota(jnp.int32, _L) ^ s
        v = v + jnp.take_along_axis(v, idx, axis=0)
    return v


def _sc_body(ps_hbm, zs_hbm, tg_hbm, out_hbm, pv, zv, tv, ones_v, zb, zcb,
             dbuf, shp, shz, shm, dma_sem):
    ci = lax.axis_index("c")
    w = lax.axis_index("s")
    g = ci * _W + w
    zero16 = jnp.zeros((_L,), jnp.float32)
    one16 = jnp.ones((_L,), jnp.float32)

    cp_p = pltpu.async_copy(ps_hbm.at[g], pv, dma_sem)
    cp_z = pltpu.async_copy(zs_hbm.at[g], zv, dma_sem)
    cp_t = pltpu.async_copy(tg_hbm.at[g], tv, dma_sem)

    for r in range(4):
        for c in range(_NCH):
            zb[r, pl.ds(c * _L, _L)] = zero16
    for r in range(5):
        for c in range(_NCH):
            zcb[r, pl.ds(c * _L, _L)] = zero16
    for c in range(_NCH):
        dbuf[pl.ds(c * _L, _L)] = zero16

    @plsc.parallel_loop(0, _H)
    def _ones_row(r):
        for c in range(_NCH):
            ones_v[r, pl.ds(c * _L, _L)] = one16

    pltpu.sync_copy(zb, shp.at[pl.ds(w * 4, 4)])
    pltpu.sync_copy(zb, shz.at[pl.ds(w * 4, 4)])
    pltpu.sync_copy(zcb, shm.at[pl.ds(w * 5, 5)])
    plsc.subcore_barrier()

    cp_p.wait()
    cp_z.wait()
    cp_t.wait()

    daccs = (zero16,) * _NCH
    scs = []
    for j in range(_B // _H):

        @plsc.parallel_loop(j * _H, (j + 1) * _H, carry=daccs)
        def daccs(r, ds_c):
            pc = [pv[r, pl.ds(c * _L, _L)] for c in range(_NCH)]
            zc = [zv[r, pl.ds(c * _L, _L)] for c in range(_NCH)]
            ssp = pc[0] * pc[0]
            ssz = zc[0] * zc[0]
            for c in range(1, _NCH):
                ssp += pc[c] * pc[c]
                ssz += zc[c] * zc[c]
            sp = _lanesum(ssp)
            sz = _lanesum(ssz)
            ip = jnp.minimum(_rsqrt16(sp), jnp.full((_L,), 1e8, jnp.float32))
            iz = jnp.minimum(_rsqrt16(sz), jnp.full((_L,), 1e8, jnp.float32))
            out = []
            for c in range(_NCH):
                pn = pc[c] * ip
                zn = zc[c] * iz
                pv[r, pl.ds(c * _L, _L)] = pn
                zv[r, pl.ds(c * _L, _L)] = zn
                out.append(ds_c[c] + pn * zn)
            return tuple(out)

        scs.append(pltpu.async_copy(
            pv.at[pl.ds(j * _H, _H)], shp.at[tv.at[j]], dma_sem, add=True))
        scs.append(pltpu.async_copy(
            zv.at[pl.ds(j * _H, _H)], shz.at[tv.at[j]], dma_sem, add=True))
        scs.append(pltpu.async_copy(
            ones_v, shm.at[tv.at[j]], dma_sem, add=True))

    dacc = daccs[0]
    for c in range(1, _NCH):
        dacc = dacc + daccs[c]
    dbuf[pl.ds(0, _L)] = dacc
    pltpu.sync_copy(dbuf, shm.at[_C + w])
    for sc in scs:
        sc.wait()
    plsc.subcore_barrier()

    q = _C // 4

    @pl.when(w < 4)
    def _dump_p():
        pltpu.sync_copy(shp.at[pl.ds(w * q, q)],
                        out_hbm.at[ci, pl.ds(w * q, q)])

    @pl.when((w >= 4) & (w < 8))
    def _dump_z():
        pltpu.sync_copy(shz.at[pl.ds((w - 4) * q, q)],
                        out_hbm.at[ci, pl.ds(_C + (w - 4) * q, q)])

    @pl.when((w >= 8) & (w < 13))
    def _dump_m():
        m = 16
        pltpu.sync_copy(shm.at[pl.ds((w - 8) * m, m)],
                        out_hbm.at[ci, pl.ds(2 * _C + (w - 8) * m, m)])


_mesh = plsc.VectorSubcoreMesh(
    core_axis_name="c", subcore_axis_name="s", num_cores=_NC, num_subcores=_W
)

_simsiam_sc = functools.partial(
    pl.kernel,
    out_type=jax.ShapeDtypeStruct((_NC, _TR, _D), jnp.float32),
    mesh=_mesh,
    compiler_params=pltpu.CompilerParams(needs_layout_passes=False),
    scratch_types=[
        pltpu.VMEM((_B, _D), jnp.float32),
        pltpu.VMEM((_B, _D), jnp.float32),
        pltpu.VMEM((_B // _H, _H), jnp.int32),
        pltpu.VMEM((_H, _D), jnp.float32),
        pltpu.VMEM((4, _D), jnp.float32),
        pltpu.VMEM((5, _D), jnp.float32),
        pltpu.VMEM((_D,), jnp.float32),
        pltpu.VMEM_SHARED((_C, _D), jnp.float32),
        pltpu.VMEM_SHARED((_C, _D), jnp.float32),
        pltpu.VMEM_SHARED((_C + _W, _D), jnp.float32),
        pltpu.SemaphoreType.DMA,
    ],
)(_sc_body)


def _tc_combine_body(t_ref, out_ref):
    sp = t_ref[0, 0:_C, :] + t_ref[1, 0:_C, :]
    sz = t_ref[0, _C:2 * _C, :] + t_ref[1, _C:2 * _C, :]
    cnt = t_ref[0, 2 * _C:3 * _C, :] + t_ref[1, 2 * _C:3 * _C, :]
    dg = t_ref[0, 3 * _C:, :] + t_ref[1, 3 * _C:, :]
    s_all = jnp.sum(sp * sz)
    d_all = jnp.sum(dg)
    nc = cnt[:, 0:1]
    pairs = jnp.sum(nc * nc) - jnp.float32(_N)
    out_ref[0, 0] = -(s_all - d_all) / pairs


_tc_combine = pl.pallas_call(
    _tc_combine_body,
    out_shape=jax.ShapeDtypeStruct((1, 1), jnp.float32),
    out_specs=pl.BlockSpec(memory_space=pltpu.MemorySpace.SMEM),
)


def kernel(ps, zs, targets):
    ps_r = ps.reshape(_NC * _W, _B, _D)
    zs_r = zs.reshape(_NC * _W, _B, _D)
    tg_r = targets.reshape(_NC * _W, _B // _H, _H).astype(jnp.int32)
    tabs = _simsiam_sc(ps_r, zs_r, tg_r)
    return _tc_combine(tabs).reshape(())


# speedup vs baseline: 1.1404x; 1.0161x over previous
import functools

import jax
import jax.numpy as jnp
from jax import lax
from jax.experimental import pallas as pl
from jax.experimental.pallas import tpu as pltpu
from jax.experimental.pallas import tpu_sc as plsc

_N = 4096
_D = 128
_C = 64
_L = 16
_NC = 2
_W = 16
_B = 128
_H = 64
_NCH = _D // _L
_TR = _C + _C + _C + _W


def _rsqrt16(x):
    x = jnp.maximum(x, jnp.full((_L,), 1e-35, jnp.float32))
    i = jnp.full((_L,), 0x5F3759DF, jnp.int32) - lax.shift_right_logical(
        plsc.bitcast(x, jnp.int32), jnp.full((_L,), 1, jnp.int32)
    )
    y = plsc.bitcast(i, jnp.float32)
    xh = x * jnp.float32(0.5)
    for _ in range(2):
        y = y * (jnp.float32(1.5) - xh * y * y)
    return y


def _lanesum(v):
    for s in (1, 2, 4, 8):
        idx = lax.iota(jnp.int32, _L) ^ s
        v = v + jnp.take_along_axis(v, idx, axis=0)
    return v


def _sc_body(ps_hbm, zs_hbm, tg_hbm, out_hbm, pv, zv, tv, ones_v, zb, zcb,
             dbuf, shp, shz, shm, dma_sem):
    ci = lax.axis_index("c")
    w = lax.axis_index("s")
    g = ci * _W + w
    zero16 = jnp.zeros((_L,), jnp.float32)
    one16 = jnp.ones((_L,), jnp.float32)

    cp0 = [pltpu.async_copy(ps_hbm.at[g, pl.ds(0, _H)],
                            pv.at[pl.ds(0, _H)], dma_sem),
           pltpu.async_copy(zs_hbm.at[g, pl.ds(0, _H)],
                            zv.at[pl.ds(0, _H)], dma_sem)]
    cp1 = [pltpu.async_copy(ps_hbm.at[g, pl.ds(_H, _H)],
                            pv.at[pl.ds(_H, _H)], dma_sem),
           pltpu.async_copy(zs_hbm.at[g, pl.ds(_H, _H)],
                            zv.at[pl.ds(_H, _H)], dma_sem),
           pltpu.async_copy(tg_hbm.at[g], tv, dma_sem)]

    for r in range(4):
        for c in range(_NCH):
            zb[r, pl.ds(c * _L, _L)] = zero16
    for r in range(5):
        for c in range(_NCH):
            zcb[r, pl.ds(c * _L, _L)] = zero16
    for c in range(_NCH):
        dbuf[pl.ds(c * _L, _L)] = zero16

    @plsc.parallel_loop(0, _H)
    def _ones_row(r):
        for c in range(_NCH):
            ones_v[r, pl.ds(c * _L, _L)] = one16

    pltpu.sync_copy(zb, shp.at[pl.ds(w * 4, 4)])
    pltpu.sync_copy(zb, shz.at[pl.ds(w * 4, 4)])
    pltpu.sync_copy(zcb, shm.at[pl.ds(w * 5, 5)])
    plsc.subcore_barrier()


    daccs = (zero16,) * _NCH
    scs = []
    for j in range(_B // _H):
        for cp in (cp0 if j == 0 else cp1):
            cp.wait()

        @plsc.parallel_loop(j * _H, (j + 1) * _H, carry=daccs)
        def daccs(r, ds_c):
            pc = [pv[r, pl.ds(c * _L, _L)] for c in range(_NCH)]
            zc = [zv[r, pl.ds(c * _L, _L)] for c in range(_NCH)]
            ssp = pc[0] * pc[0]
            ssz = zc[0] * zc[0]
            for c in range(1, _NCH):
                ssp += pc[c] * pc[c]
                ssz += zc[c] * zc[c]
            sp = _lanesum(ssp)
            sz = _lanesum(ssz)
            ip = jnp.minimum(_rsqrt16(sp), jnp.full((_L,), 1e8, jnp.float32))
            iz = jnp.minimum(_rsqrt16(sz), jnp.full((_L,), 1e8, jnp.float32))
            out = []
            for c in range(_NCH):
                pn = pc[c] * ip
                zn = zc[c] * iz
                pv[r, pl.ds(c * _L, _L)] = pn
                zv[r, pl.ds(c * _L, _L)] = zn
                out.append(ds_c[c] + pn * zn)
            return tuple(out)

        scs.append(pltpu.async_copy(
            pv.at[pl.ds(j * _H, _H)], shp.at[tv.at[j]], dma_sem, add=True))
        scs.append(pltpu.async_copy(
            zv.at[pl.ds(j * _H, _H)], shz.at[tv.at[j]], dma_sem, add=True))
        scs.append(pltpu.async_copy(
            ones_v, shm.at[tv.at[j]], dma_sem, add=True))

    dacc = daccs[0]
    for c in range(1, _NCH):
        dacc = dacc + daccs[c]
    dbuf[pl.ds(0, _L)] = dacc
    pltpu.sync_copy(dbuf, shm.at[_C + w])
    for sc in scs:
        sc.wait()
    plsc.subcore_barrier()

    q = _C // 4

    @pl.when(w < 4)
    def _dump_p():
        pltpu.sync_copy(shp.at[pl.ds(w * q, q)],
                        out_hbm.at[ci, pl.ds(w * q, q)])

    @pl.when((w >= 4) & (w < 8))
    def _dump_z():
        pltpu.sync_copy(shz.at[pl.ds((w - 4) * q, q)],
                        out_hbm.at[ci, pl.ds(_C + (w - 4) * q, q)])

    @pl.when((w >= 8) & (w < 13))
    def _dump_m():
        m = 16
        pltpu.sync_copy(shm.at[pl.ds((w - 8) * m, m)],
                        out_hbm.at[ci, pl.ds(2 * _C + (w - 8) * m, m)])


_mesh = plsc.VectorSubcoreMesh(
    core_axis_name="c", subcore_axis_name="s", num_cores=_NC, num_subcores=_W
)

_simsiam_sc = functools.partial(
    pl.kernel,
    out_type=jax.ShapeDtypeStruct((_NC, _TR, _D), jnp.float32),
    mesh=_mesh,
    compiler_params=pltpu.CompilerParams(needs_layout_passes=False),
    scratch_types=[
        pltpu.VMEM((_B, _D), jnp.float32),
        pltpu.VMEM((_B, _D), jnp.float32),
        pltpu.VMEM((_B // _H, _H), jnp.int32),
        pltpu.VMEM((_H, _D), jnp.float32),
        pltpu.VMEM((4, _D), jnp.float32),
        pltpu.VMEM((5, _D), jnp.float32),
        pltpu.VMEM((_D,), jnp.float32),
        pltpu.VMEM_SHARED((_C, _D), jnp.float32),
        pltpu.VMEM_SHARED((_C, _D), jnp.float32),
        pltpu.VMEM_SHARED((_C + _W, _D), jnp.float32),
        pltpu.SemaphoreType.DMA,
    ],
)(_sc_body)


def _tc_combine_body(t_ref, out_ref):
    sp = t_ref[0, 0:_C, :] + t_ref[1, 0:_C, :]
    sz = t_ref[0, _C:2 * _C, :] + t_ref[1, _C:2 * _C, :]
    cnt = t_ref[0, 2 * _C:3 * _C, :] + t_ref[1, 2 * _C:3 * _C, :]
    dg = t_ref[0, 3 * _C:, :] + t_ref[1, 3 * _C:, :]
    s_all = jnp.sum(sp * sz)
    d_all = jnp.sum(dg)
    nc = cnt[:, 0:1]
    pairs = jnp.sum(nc * nc) - jnp.float32(_N)
    out_ref[0, 0] = -(s_all - d_all) / pairs


_tc_combine = pl.pallas_call(
    _tc_combine_body,
    out_shape=jax.ShapeDtypeStruct((1, 1), jnp.float32),
    out_specs=pl.BlockSpec(memory_space=pltpu.MemorySpace.SMEM),
)


def kernel(ps, zs, targets):
    ps_r = ps.reshape(_NC * _W, _B, _D)
    zs_r = zs.reshape(_NC * _W, _B, _D)
    tg_r = targets.reshape(_NC * _W, _B // _H, _H).astype(jnp.int32)
    tabs = _simsiam_sc(ps_r, zs_r, tg_r)
    return _tc_combine(tabs).reshape(())
